# baseline (device time: 29395 ns/iter reference)
import jax
import jax.numpy as jnp
from jax import lax
from jax.experimental import pallas as pl
from jax.experimental.pallas import tpu as pltpu

T_LOC = 256
D = 512
E_LOC = 2
F = 1024
HALF = T_LOC // 2
NCH = 2
CH = HALF // NCH


def _top2_weights(gates):
    m1 = jnp.max(gates, axis=1, keepdims=True)
    mask1 = gates >= m1
    rest = jnp.where(mask1, -jnp.inf, gates)
    m2 = jnp.max(rest, axis=1, keepdims=True)
    mask2 = rest >= m2
    z = jnp.exp(m2 - m1)
    w1v = 1.0 / (1.0 + z)
    w2v = z / (1.0 + z)
    return w1v * mask1.astype(jnp.float32) + w2v * mask2.astype(jnp.float32)


def _expert_block(x_blk, router_mine, router_other, w1_ref, w2_ref):
    gates = jnp.concatenate([
        jnp.dot(x_blk, router_mine, preferred_element_type=jnp.float32,
                precision=lax.Precision.HIGHEST),
        jnp.dot(x_blk, router_other, preferred_element_type=jnp.float32,
                precision=lax.Precision.HIGHEST),
    ], axis=1)
    wgt = _top2_weights(gates)
    acc = jnp.zeros((x_blk.shape[0], D), jnp.float32)
    for l in range(E_LOC):
        h = jnp.maximum(
            jnp.dot(x_blk, w1_ref[l], preferred_element_type=jnp.float32),
            0.0)
        y_e = jnp.dot(h, w2_ref[l], preferred_element_type=jnp.float32)
        acc = acc + y_e * wgt[:, l:l + 1]
    return acc


def kernel(x, router, W1, W2):
    def body(x_ref, router_ref, w1_hbm, w2_hbm, out_ref,
             x_send, x_other, router_other, part_send, comb_direct,
             comb_fwd, w1_ref, w2_ref, disp_s, disp_r, comb_s, comb_r,
             fwd_s, fwd_r, rtr_s, rtr_r, w_sems):
        my_x = lax.axis_index("x")
        my_y = lax.axis_index("y")
        ypeer = (my_x, 1 - my_y)
        xnbr = (1 - my_x, my_y)

        w1_dma = pltpu.make_async_copy(w1_hbm, w1_ref, w_sems.at[0])
        w1_dma.start()
        w2_dma = pltpu.make_async_copy(w2_hbm, w2_ref, w_sems.at[1])
        w2_dma.start()

        bsem = pltpu.get_barrier_semaphore()
        pl.semaphore_signal(bsem, inc=1, device_id=my_x * 2 + (1 - my_y),
                            device_id_type=pl.DeviceIdType.LOGICAL)
        pl.semaphore_signal(bsem, inc=1, device_id=(1 - my_x) * 2 + my_y,
                            device_id_type=pl.DeviceIdType.LOGICAL)
        pl.semaphore_wait(bsem, 2)

        xm = x_ref[:, :]
        x_send[:, :] = jnp.where(my_x == 0, xm[:HALF], xm[HALF:])
        rdma_disp = []
        for c in range(NCH):
            r = pltpu.make_async_remote_copy(
                src_ref=x_send.at[pl.ds(c * CH, CH)],
                dst_ref=x_other.at[c],
                send_sem=disp_s.at[c], recv_sem=disp_r.at[c],
                device_id=ypeer, device_id_type=pl.DeviceIdType.MESH,
            )
            r.start()
            rdma_disp.append(r)
        rdma_rtr = pltpu.make_async_remote_copy(
            src_ref=router_ref, dst_ref=router_other,
            send_sem=rtr_s, recv_sem=rtr_r,
            device_id=ypeer, device_id_type=pl.DeviceIdType.MESH,
        )
        rdma_rtr.start()
        rdma_rtr.wait()

        w1_dma.wait()
        w2_dma.wait()
        r_mine = router_ref[:, :]
        r_other = router_other[:, :]
        acc_mine = _expert_block(xm, r_mine, r_other, w1_ref, w2_ref)

        rdma_comb = []
        for c in range(NCH):
            rdma_disp[c].wait()
            part_send[c] = _expert_block(x_other[c], r_mine, r_other,
                                         w1_ref, w2_ref)
            r = pltpu.make_async_remote_copy(
                src_ref=part_send.at[c], dst_ref=comb_direct.at[c],
                send_sem=comb_s.at[c], recv_sem=comb_r.at[c],
                device_id=ypeer, device_id_type=pl.DeviceIdType.MESH,
            )
            r.start()
            rdma_comb.append(r)

        rdma_fwd = []
        for c in range(NCH):
            rdma_comb[c].wait()
            r = pltpu.make_async_remote_copy(
                src_ref=comb_direct.at[c], dst_ref=comb_fwd.at[c],
                send_sem=fwd_s.at[c], recv_sem=fwd_r.at[c],
                device_id=xnbr, device_id_type=pl.DeviceIdType.MESH,
            )
            r.start()
            rdma_fwd.append(r)

            @pl.when(my_x == 0)
            def _():
                out_ref[c * CH:(c + 1) * CH, :] = (
                    acc_mine[c * CH:(c + 1) * CH] + comb_direct[c])

            @pl.when(my_x == 1)
            def _():
                out_ref[HALF + c * CH:HALF + (c + 1) * CH, :] = (
                    acc_mine[HALF + c * CH:HALF + (c + 1) * CH]
                    + comb_direct[c])

        for c in range(NCH):
            rdma_fwd[c].wait()

            @pl.when(my_x == 0)
            def _():
                out_ref[HALF + c * CH:HALF + (c + 1) * CH, :] = (
                    acc_mine[HALF + c * CH:HALF + (c + 1) * CH]
                    + comb_fwd[c])

            @pl.when(my_x == 1)
            def _():
                out_ref[c * CH:(c + 1) * CH, :] = (
                    acc_mine[c * CH:(c + 1) * CH] + comb_fwd[c])

    return pl.pallas_call(
        body,
        out_shape=jax.ShapeDtypeStruct((T_LOC, D), jnp.float32),
        in_specs=[
            pl.BlockSpec(memory_space=pltpu.VMEM),
            pl.BlockSpec(memory_space=pltpu.VMEM),
            pl.BlockSpec(memory_space=pl.ANY),
            pl.BlockSpec(memory_space=pl.ANY),
        ],
        out_specs=pl.BlockSpec(memory_space=pltpu.VMEM),
        scratch_shapes=[
            pltpu.VMEM((HALF, D), jnp.float32),
            pltpu.VMEM((NCH, CH, D), jnp.float32),
            pltpu.VMEM((D, E_LOC), jnp.float32),
            pltpu.VMEM((NCH, CH, D), jnp.float32),
            pltpu.VMEM((NCH, CH, D), jnp.float32),
            pltpu.VMEM((NCH, CH, D), jnp.float32),
            pltpu.VMEM((E_LOC, D, F), jnp.float32),
            pltpu.VMEM((E_LOC, F, D), jnp.float32),
            pltpu.SemaphoreType.DMA((NCH,)),
            pltpu.SemaphoreType.DMA((NCH,)),
            pltpu.SemaphoreType.DMA((NCH,)),
            pltpu.SemaphoreType.DMA((NCH,)),
            pltpu.SemaphoreType.DMA((NCH,)),
            pltpu.SemaphoreType.DMA((NCH,)),
            pltpu.SemaphoreType.DMA,
            pltpu.SemaphoreType.DMA,
            pltpu.SemaphoreType.DMA((2,)),
        ],
        compiler_params=pltpu.CompilerParams(collective_id=0),
    )(x, router, W1, W2)


# device time: 28665 ns/iter; 1.0255x vs baseline; 1.0255x over previous
import jax
import jax.numpy as jnp
from jax import lax
from jax.experimental import pallas as pl
from jax.experimental.pallas import tpu as pltpu

T_LOC = 256
D = 512
E_LOC = 2
F = 1024
HALF = T_LOC // 2
NCH = 4
CH = HALF // NCH


def _top2_weights(gates):
    m1 = jnp.max(gates, axis=1, keepdims=True)
    mask1 = gates >= m1
    rest = jnp.where(mask1, -jnp.inf, gates)
    m2 = jnp.max(rest, axis=1, keepdims=True)
    mask2 = rest >= m2
    z = jnp.exp(m2 - m1)
    w1v = 1.0 / (1.0 + z)
    w2v = z / (1.0 + z)
    return w1v * mask1.astype(jnp.float32) + w2v * mask2.astype(jnp.float32)


def _routing(x_blk, router_mine, router_other):
    gates = jnp.concatenate([
        jnp.dot(x_blk, router_mine, preferred_element_type=jnp.float32,
                precision=lax.Precision.HIGHEST),
        jnp.dot(x_blk, router_other, preferred_element_type=jnp.float32,
                precision=lax.Precision.HIGHEST),
    ], axis=1)
    return _top2_weights(gates)


def _expert_contrib(x_blk, wgt, l, w1_ref, w2_ref):
    h = jnp.maximum(
        jnp.dot(x_blk, w1_ref[l], preferred_element_type=jnp.float32), 0.0)
    y_e = jnp.dot(h, w2_ref[l], preferred_element_type=jnp.float32)
    return y_e * wgt[:, l:l + 1]


def _expert_block(x_blk, router_mine, router_other, w1_ref, w2_ref):
    wgt = _routing(x_blk, router_mine, router_other)
    acc = jnp.zeros((x_blk.shape[0], D), jnp.float32)
    for l in range(E_LOC):
        acc = acc + _expert_contrib(x_blk, wgt, l, w1_ref, w2_ref)
    return acc


def kernel(x, router, W1, W2):
    def body(x_ref, router_ref, w1_ref, w2_ref, out_ref,
             x_send, x_other, router_other, part_send, comb_direct,
             comb_fwd, disp_s, disp_r, comb_s, comb_r, fwd_s, fwd_r,
             rtr_s, rtr_r):
        my_x = lax.axis_index("x")
        my_y = lax.axis_index("y")
        ypeer = (my_x, 1 - my_y)
        xnbr = (1 - my_x, my_y)

        bsem = pltpu.get_barrier_semaphore()
        pl.semaphore_signal(bsem, inc=1, device_id=my_x * 2 + (1 - my_y),
                            device_id_type=pl.DeviceIdType.LOGICAL)
        pl.semaphore_signal(bsem, inc=1, device_id=(1 - my_x) * 2 + my_y,
                            device_id_type=pl.DeviceIdType.LOGICAL)
        pl.semaphore_wait(bsem, 2)

        xm = x_ref[:, :]
        x_send[:, :] = jnp.where(my_x == 0, xm[:HALF], xm[HALF:])
        rdma_disp = []
        for c in range(NCH):
            r = pltpu.make_async_remote_copy(
                src_ref=x_send.at[pl.ds(c * CH, CH)],
                dst_ref=x_other.at[c],
                send_sem=disp_s.at[c], recv_sem=disp_r.at[c],
                device_id=ypeer, device_id_type=pl.DeviceIdType.MESH,
            )
            r.start()
            rdma_disp.append(r)
        rdma_rtr = pltpu.make_async_remote_copy(
            src_ref=router_ref, dst_ref=router_other,
            send_sem=rtr_s, recv_sem=rtr_r,
            device_id=ypeer, device_id_type=pl.DeviceIdType.MESH,
        )
        rdma_rtr.start()
        rdma_rtr.wait()

        r_mine = router_ref[:, :]
        r_other = router_other[:, :]
        wgt_mine = _routing(xm, r_mine, r_other)
        acc_mine = _expert_contrib(xm, wgt_mine, 0, w1_ref, w2_ref)

        rdma_comb = []
        for c in range(NCH):
            rdma_disp[c].wait()
            part_send[c] = _expert_block(x_other[c], r_mine, r_other,
                                         w1_ref, w2_ref)
            r = pltpu.make_async_remote_copy(
                src_ref=part_send.at[c], dst_ref=comb_direct.at[c],
                send_sem=comb_s.at[c], recv_sem=comb_r.at[c],
                device_id=ypeer, device_id_type=pl.DeviceIdType.MESH,
            )
            r.start()
            rdma_comb.append(r)

        acc_mine = acc_mine + _expert_contrib(xm, wgt_mine, 1,
                                              w1_ref, w2_ref)

        rdma_fwd = []
        for c in range(NCH):
            rdma_comb[c].wait()
            r = pltpu.make_async_remote_copy(
                src_ref=comb_direct.at[c], dst_ref=comb_fwd.at[c],
                send_sem=fwd_s.at[c], recv_sem=fwd_r.at[c],
                device_id=xnbr, device_id_type=pl.DeviceIdType.MESH,
            )
            r.start()
            rdma_fwd.append(r)

        for c in range(NCH):
            @pl.when(my_x == 0)
            def _():
                out_ref[c * CH:(c + 1) * CH, :] = (
                    acc_mine[c * CH:(c + 1) * CH] + comb_direct[c])

            @pl.when(my_x == 1)
            def _():
                out_ref[HALF + c * CH:HALF + (c + 1) * CH, :] = (
                    acc_mine[HALF + c * CH:HALF + (c + 1) * CH]
                    + comb_direct[c])

        for c in range(NCH):
            rdma_fwd[c].wait()

            @pl.when(my_x == 0)
            def _():
                out_ref[HALF + c * CH:HALF + (c + 1) * CH, :] = (
                    acc_mine[HALF + c * CH:HALF + (c + 1) * CH]
                    + comb_fwd[c])

            @pl.when(my_x == 1)
            def _():
                out_ref[c * CH:(c + 1) * CH, :] = (
                    acc_mine[c * CH:(c + 1) * CH] + comb_fwd[c])

    return pl.pallas_call(
        body,
        out_shape=jax.ShapeDtypeStruct((T_LOC, D), jnp.float32),
        in_specs=[pl.BlockSpec(memory_space=pltpu.VMEM)] * 4,
        out_specs=pl.BlockSpec(memory_space=pltpu.VMEM),
        scratch_shapes=[
            pltpu.VMEM((HALF, D), jnp.float32),
            pltpu.VMEM((NCH, CH, D), jnp.float32),
            pltpu.VMEM((D, E_LOC), jnp.float32),
            pltpu.VMEM((NCH, CH, D), jnp.float32),
            pltpu.VMEM((NCH, CH, D), jnp.float32),
            pltpu.VMEM((NCH, CH, D), jnp.float32),
            pltpu.SemaphoreType.DMA((NCH,)),
            pltpu.SemaphoreType.DMA((NCH,)),
            pltpu.SemaphoreType.DMA((NCH,)),
            pltpu.SemaphoreType.DMA((NCH,)),
            pltpu.SemaphoreType.DMA((NCH,)),
            pltpu.SemaphoreType.DMA((NCH,)),
            pltpu.SemaphoreType.DMA,
            pltpu.SemaphoreType.DMA,
        ],
        compiler_params=pltpu.CompilerParams(collective_id=0),
    )(x, router, W1, W2)


# device time: 28219 ns/iter; 1.0417x vs baseline; 1.0158x over previous
import jax
import jax.numpy as jnp
from jax import lax
from jax.experimental import pallas as pl
from jax.experimental.pallas import tpu as pltpu

T_LOC = 256
D = 512
E_LOC = 2
F = 1024
HALF = T_LOC // 2
NCH = 2
CH = HALF // NCH


def _top2_weights(gates):
    m1 = jnp.max(gates, axis=1, keepdims=True)
    mask1 = gates >= m1
    rest = jnp.where(mask1, -jnp.inf, gates)
    m2 = jnp.max(rest, axis=1, keepdims=True)
    mask2 = rest >= m2
    z = jnp.exp(m2 - m1)
    w1v = 1.0 / (1.0 + z)
    w2v = z / (1.0 + z)
    return w1v * mask1.astype(jnp.float32) + w2v * mask2.astype(jnp.float32)


def _routing(x_blk, router_mine, router_other):
    gates = jnp.concatenate([
        jnp.dot(x_blk, router_mine, preferred_element_type=jnp.float32,
                precision=lax.Precision.HIGHEST),
        jnp.dot(x_blk, router_other, preferred_element_type=jnp.float32,
                precision=lax.Precision.HIGHEST),
    ], axis=1)
    return _top2_weights(gates)


def _expert_contrib(x_blk, wgt, l, w1_ref, w2_ref):
    h = jnp.maximum(
        jnp.dot(x_blk, w1_ref[l], preferred_element_type=jnp.float32), 0.0)
    y_e = jnp.dot(h, w2_ref[l], preferred_element_type=jnp.float32)
    return y_e * wgt[:, l:l + 1]


def _expert_block(x_blk, router_mine, router_other, w1_ref, w2_ref):
    wgt = _routing(x_blk, router_mine, router_other)
    acc = jnp.zeros((x_blk.shape[0], D), jnp.float32)
    for l in range(E_LOC):
        acc = acc + _expert_contrib(x_blk, wgt, l, w1_ref, w2_ref)
    return acc


def kernel(x, router, W1, W2):
    def body(x_ref, router_ref, w1_ref, w2_ref, out_ref,
             x_send, x_other, router_other, part_send, comb_direct,
             comb_fwd, disp_s, disp_r, comb_s, comb_r, fwd_s, fwd_r,
             rtr_s, rtr_r):
        my_x = lax.axis_index("x")
        my_y = lax.axis_index("y")
        ypeer = (my_x, 1 - my_y)
        xnbr = (1 - my_x, my_y)

        bsem = pltpu.get_barrier_semaphore()
        pl.semaphore_signal(bsem, inc=1, device_id=my_x * 2 + (1 - my_y),
                            device_id_type=pl.DeviceIdType.LOGICAL)
        pl.semaphore_signal(bsem, inc=1, device_id=(1 - my_x) * 2 + my_y,
                            device_id_type=pl.DeviceIdType.LOGICAL)
        pl.semaphore_wait(bsem, 2)

        xm = x_ref[:, :]
        x_send[:, :] = jnp.where(my_x == 0, xm[:HALF], xm[HALF:])
        rdma_disp = []
        for c in range(NCH):
            r = pltpu.make_async_remote_copy(
                src_ref=x_send.at[pl.ds(c * CH, CH)],
                dst_ref=x_other.at[c],
                send_sem=disp_s.at[c], recv_sem=disp_r.at[c],
                device_id=ypeer, device_id_type=pl.DeviceIdType.MESH,
            )
            r.start()
            rdma_disp.append(r)
        rdma_rtr = pltpu.make_async_remote_copy(
            src_ref=router_ref, dst_ref=router_other,
            send_sem=rtr_s, recv_sem=rtr_r,
            device_id=ypeer, device_id_type=pl.DeviceIdType.MESH,
        )
        rdma_rtr.start()
        rdma_rtr.wait()

        r_mine = router_ref[:, :]
        r_other = router_other[:, :]
        wgt_mine = _routing(xm, r_mine, r_other)
        acc_mine = _expert_contrib(xm, wgt_mine, 0, w1_ref, w2_ref)

        rdma_comb = []
        for c in range(NCH):
            rdma_disp[c].wait()
            part_send[c] = _expert_block(x_other[c], r_mine, r_other,
                                         w1_ref, w2_ref)
            r = pltpu.make_async_remote_copy(
                src_ref=part_send.at[c], dst_ref=comb_direct.at[c],
                send_sem=comb_s.at[c], recv_sem=comb_r.at[c],
                device_id=ypeer, device_id_type=pl.DeviceIdType.MESH,
            )
            r.start()
            rdma_comb.append(r)

        acc_mine = acc_mine + _expert_contrib(xm, wgt_mine, 1,
                                              w1_ref, w2_ref)

        rdma_fwd = []
        for c in range(NCH):
            rdma_comb[c].wait()
            r = pltpu.make_async_remote_copy(
                src_ref=comb_direct.at[c], dst_ref=comb_fwd.at[c],
                send_sem=fwd_s.at[c], recv_sem=fwd_r.at[c],
                device_id=xnbr, device_id_type=pl.DeviceIdType.MESH,
            )
            r.start()
            rdma_fwd.append(r)

        for c in range(NCH):
            @pl.when(my_x == 0)
            def _():
                out_ref[c * CH:(c + 1) * CH, :] = (
                    acc_mine[c * CH:(c + 1) * CH] + comb_direct[c])

            @pl.when(my_x == 1)
            def _():
                out_ref[HALF + c * CH:HALF + (c + 1) * CH, :] = (
                    acc_mine[HALF + c * CH:HALF + (c + 1) * CH]
                    + comb_direct[c])

        for c in range(NCH):
            rdma_fwd[c].wait()

            @pl.when(my_x == 0)
            def _():
                out_ref[HALF + c * CH:HALF + (c + 1) * CH, :] = (
                    acc_mine[HALF + c * CH:HALF + (c + 1) * CH]
                    + comb_fwd[c])

            @pl.when(my_x == 1)
            def _():
                out_ref[c * CH:(c + 1) * CH, :] = (
                    acc_mine[c * CH:(c + 1) * CH] + comb_fwd[c])

    return pl.pallas_call(
        body,
        out_shape=jax.ShapeDtypeStruct((T_LOC, D), jnp.float32),
        in_specs=[pl.BlockSpec(memory_space=pltpu.VMEM)] * 4,
        out_specs=pl.BlockSpec(memory_space=pltpu.VMEM),
        scratch_shapes=[
            pltpu.VMEM((HALF, D), jnp.float32),
            pltpu.VMEM((NCH, CH, D), jnp.float32),
            pltpu.VMEM((D, E_LOC), jnp.float32),
            pltpu.VMEM((NCH, CH, D), jnp.float32),
            pltpu.VMEM((NCH, CH, D), jnp.float32),
            pltpu.VMEM((NCH, CH, D), jnp.float32),
            pltpu.SemaphoreType.DMA((NCH,)),
            pltpu.SemaphoreType.DMA((NCH,)),
            pltpu.SemaphoreType.DMA((NCH,)),
            pltpu.SemaphoreType.DMA((NCH,)),
            pltpu.SemaphoreType.DMA((NCH,)),
            pltpu.SemaphoreType.DMA((NCH,)),
            pltpu.SemaphoreType.DMA,
            pltpu.SemaphoreType.DMA,
        ],
        compiler_params=pltpu.CompilerParams(collective_id=0),
    )(x, router, W1, W2)
